# Initial kernel scaffold; baseline (speedup 1.0000x reference)
import jax
import jax.numpy as jnp
from jax.experimental import pallas as pl


def kernel(feature, coord, W0, b0, g0, be0, W1, b1, g1, be1):
    raise NotImplementedError("write your pallas kernel here")


# speedup vs baseline: 10.8728x; 10.8728x over previous
import functools

import jax
import jax.numpy as jnp
from jax import lax
from jax.experimental import pallas as pl
from jax.experimental.pallas import tpu as pltpu
from jax.experimental.pallas import tpu_sc as plsc

B = 8
N = 4096
G = 1024
K = 16
DF = 128
DX = 256
DC = 144
D1 = 256
D2 = 256
P = B * G * K

NC, NS = 2, 16
NW = NC * NS
BPW = P // NW
SC_CH = 128
SC_NCH = BPW // SC_CH


def _fps_body(xr, yr, zr, cidx_r, cx_r, cy_r, cz_r, d_ref):
    iota_n = lax.broadcasted_iota(jnp.int32, (B, N), 1)
    iota_g = lax.broadcasted_iota(jnp.int32, (B, G), 1)
    d_ref[...] = jnp.full((B, N), 1e10, jnp.float32)
    cidx_r[...] = jnp.zeros((B, G), jnp.int32)
    cx_r[...] = jnp.zeros((B, G), jnp.float32)
    cy_r[...] = jnp.zeros((B, G), jnp.float32)
    cz_r[...] = jnp.zeros((B, G), jnp.float32)
    X = xr[...]
    Y = yr[...]
    Z = zr[...]

    def step(i, far):
        sel = iota_g == i
        m2 = iota_n == far
        cx = jnp.sum(jnp.where(m2, X, 0.0), 1, keepdims=True)
        cy = jnp.sum(jnp.where(m2, Y, 0.0), 1, keepdims=True)
        cz = jnp.sum(jnp.where(m2, Z, 0.0), 1, keepdims=True)
        cidx_r[...] = jnp.where(sel, far, cidx_r[...])
        cx_r[...] = jnp.where(sel, cx, cx_r[...])
        cy_r[...] = jnp.where(sel, cy, cy_r[...])
        cz_r[...] = jnp.where(sel, cz, cz_r[...])
        dx = X - cx
        dy = Y - cy
        dz = Z - cz
        dist = (dx * dx + dz * dz) + dy * dy
        D = jnp.minimum(d_ref[...], dist)
        d_ref[...] = D
        mx = jnp.max(D, 1, keepdims=True)
        farn = jnp.min(jnp.where(D == mx, iota_n, N), 1, keepdims=True)
        return farn

    lax.fori_loop(0, G, step, jnp.zeros((B, 1), jnp.int32))


def _run_fps(xp, yp, zp):
    return pl.pallas_call(
        _fps_body,
        grid=(1,),
        in_specs=[pl.BlockSpec((B, N), lambda i: (0, 0))] * 3,
        out_specs=[pl.BlockSpec((B, G), lambda i: (0, 0))] * 4,
        out_shape=[
            jax.ShapeDtypeStruct((B, G), jnp.int32),
            jax.ShapeDtypeStruct((B, G), jnp.float32),
            jax.ShapeDtypeStruct((B, G), jnp.float32),
            jax.ShapeDtypeStruct((B, G), jnp.float32),
        ],
        scratch_shapes=[pltpu.VMEM((B, N), jnp.float32)],
    )(xp, yp, zp)


GT = 128


def _knn_body(q_r, p_r, out_r, dist_r):
    b = pl.program_id(0)
    q = q_r[0]
    p = p_r[0]
    qp = lax.dot_general(q, p, (((1,), (0,)), ((), ())),
                         preferred_element_type=jnp.float32)
    px = p[0:1, :]; py = p[1:2, :]; pz = p[2:3, :]
    qx = q[:, 0:1]; qy = q[:, 1:2]; qz = q[:, 2:3]
    sq_p = px * px + py * py + pz * pz
    sq_c = qx * qx + qy * qy + qz * qz
    dist_r[...] = sq_c + sq_p - 2.0 * qp
    iota = lax.broadcasted_iota(jnp.int32, (GT, N), 1)
    base = b * N
    dist = dist_r[...]
    m_prev = jnp.full((GT, 1), -jnp.inf, jnp.float32)
    s_prev = jnp.full((GT, 1), -1, jnp.int32)
    for j in range(K):
        ok = (dist > m_prev) | ((dist == m_prev) & (iota > s_prev))
        cand = jnp.where(ok, dist, jnp.inf)
        m = jnp.min(cand, 1, keepdims=True)
        s = jnp.min(jnp.where(cand == m, iota, N), 1, keepdims=True)
        out_r[0, :, j:j + 1] = s + base
        m_prev, s_prev = m, s


def _run_knn(qpad, ppad):
    return pl.pallas_call(
        _knn_body,
        grid=(B, G // GT),
        in_specs=[
            pl.BlockSpec((1, GT, 4), lambda b, g: (b, g, 0)),
            pl.BlockSpec((1, 4, N), lambda b, g: (b, 0, 0)),
        ],
        out_specs=pl.BlockSpec((1, GT, K), lambda b, g: (b, g, 0)),
        out_shape=jax.ShapeDtypeStruct((B, G, K), jnp.int32),
        scratch_shapes=[pltpu.VMEM((GT, N), jnp.float32)],
    )(qpad, ppad)


def _sc_gather_body(tbl_hbm, idx_hbm, out_hbm, idx_v, rows_v, sem):
    wid = lax.axis_index("s") * NC + lax.axis_index("c")
    row0 = wid * SC_NCH
    pltpu.sync_copy(idx_hbm.at[pl.ds(row0, SC_NCH)], idx_v)

    def chunk(i, carry):
        pltpu.async_copy(tbl_hbm.at[idx_v.at[i]], rows_v, sem).wait()
        pltpu.sync_copy(rows_v, out_hbm.at[pl.ds(wid * BPW + i * SC_CH, SC_CH)])
        return carry

    lax.fori_loop(0, SC_NCH, chunk, 0)


def _run_sc_gather(tbl, idx2d):
    mesh = plsc.VectorSubcoreMesh(core_axis_name="c", subcore_axis_name="s")
    f = functools.partial(
        pl.kernel,
        mesh=mesh,
        out_type=jax.ShapeDtypeStruct((P, DX), jnp.float32),
        scratch_types=[
            pltpu.VMEM((SC_NCH, SC_CH), jnp.int32),
            pltpu.VMEM((SC_CH, DX), jnp.float32),
            pltpu.SemaphoreType.DMA,
        ],
    )(_sc_gather_body)
    return f(tbl, idx2d)


PT = 512
NPT = P // PT


def _stats_body(g_r, c_r, m_r, s_r):
    xf = g_r[:, 0:DF]
    xc = g_r[:, DF:DC] - c_r[...]
    X = jnp.concatenate([xf, xc], axis=1)

    @pl.when(pl.program_id(0) == 0)
    def _():
        m_r[...] = jnp.zeros((DC, DC), jnp.float32)
        s_r[...] = jnp.zeros((1, DC), jnp.float32)

    m_r[...] += lax.dot_general(X, X, (((0,), (0,)), ((), ())),
                                preferred_element_type=jnp.float32)
    s_r[...] += jnp.sum(X, 0, keepdims=True)


def _run_stats(gath, crep):
    return pl.pallas_call(
        _stats_body,
        grid=(NPT,),
        in_specs=[
            pl.BlockSpec((PT, DX), lambda i: (i, 0)),
            pl.BlockSpec((PT, K), lambda i: (i, 0)),
        ],
        out_specs=[
            pl.BlockSpec((DC, DC), lambda i: (0, 0)),
            pl.BlockSpec((1, DC), lambda i: (0, 0)),
        ],
        out_shape=[
            jax.ShapeDtypeStruct((DC, DC), jnp.float32),
            jax.ShapeDtypeStruct((1, DC), jnp.float32),
        ],
    )(gath, crep)


def _mlp_body(g_r, c_r, w0_r, b0_r, w1_r, b1_r, km_r, ss_r, sq_r):
    xf = g_r[:, 0:DF]
    xc = g_r[:, DF:DC] - c_r[...]
    X = jnp.concatenate([xf, xc], axis=1)
    y1 = lax.dot_general(X, w0_r[...], (((1,), (0,)), ((), ())),
                         preferred_element_type=jnp.float32) + b0_r[...]
    x2 = jnp.maximum(y1, 0.0)
    y2 = lax.dot_general(x2, w1_r[...], (((1,), (0,)), ((), ())),
                         preferred_element_type=jnp.float32) + b1_r[...]

    @pl.when(pl.program_id(0) == 0)
    def _():
        ss_r[...] = jnp.zeros((1, D2), jnp.float32)
        sq_r[...] = jnp.zeros((1, D2), jnp.float32)

    ss_r[...] += jnp.sum(y2, 0, keepdims=True)
    sq_r[...] += jnp.sum(y2 * y2, 0, keepdims=True)
    km_r[...] = jnp.max(y2.reshape(PT // K, K, D2), axis=1)


def _run_mlp(gath, crep, w0s, b0s, w1t, b1):
    return pl.pallas_call(
        _mlp_body,
        grid=(NPT,),
        in_specs=[
            pl.BlockSpec((PT, DX), lambda i: (i, 0)),
            pl.BlockSpec((PT, K), lambda i: (i, 0)),
            pl.BlockSpec((DC, D1), lambda i: (0, 0)),
            pl.BlockSpec((1, D1), lambda i: (0, 0)),
            pl.BlockSpec((D1, D2), lambda i: (0, 0)),
            pl.BlockSpec((1, D2), lambda i: (0, 0)),
        ],
        out_specs=[
            pl.BlockSpec((PT // K, D2), lambda i: (i, 0)),
            pl.BlockSpec((1, D2), lambda i: (0, 0)),
            pl.BlockSpec((1, D2), lambda i: (0, 0)),
        ],
        out_shape=[
            jax.ShapeDtypeStruct((B * G, D2), jnp.float32),
            jax.ShapeDtypeStruct((1, D2), jnp.float32),
            jax.ShapeDtypeStruct((1, D2), jnp.float32),
        ],
    )(gath, crep, w0s, b0s, w1t, b1)


FT = 2048


def _final_body(km_r, a_r, c_r, o_r):
    o_r[...] = jnp.maximum(a_r[...] * km_r[...] + c_r[...], 0.0)


def _run_final(km, a2, c2):
    return pl.pallas_call(
        _final_body,
        grid=(B * G // FT,),
        in_specs=[
            pl.BlockSpec((FT, D2), lambda i: (i, 0)),
            pl.BlockSpec((1, D2), lambda i: (0, 0)),
            pl.BlockSpec((1, D2), lambda i: (0, 0)),
        ],
        out_specs=pl.BlockSpec((FT, D2), lambda i: (i, 0)),
        out_shape=jax.ShapeDtypeStruct((B * G, D2), jnp.float32),
    )(km, a2, c2)


def kernel(feature, coord, W0, b0, g0, be0, W1, b1, g1, be1):
    xp = coord[..., 0]
    yp = coord[..., 1]
    zp = coord[..., 2]
    cidx, cx, cy, cz = _run_fps(xp, yp, zp)
    del cidx
    center_coord = jnp.stack([cx, cy, cz], axis=-1)

    qpad = jnp.stack([cx, cy, cz, jnp.zeros_like(cx)], axis=-1)
    ppad = jnp.concatenate(
        [coord.transpose(0, 2, 1),
         jnp.zeros((B, 1, N), jnp.float32)], axis=1)
    nidx = _run_knn(qpad, ppad)

    tbl = jnp.concatenate(
        [feature.reshape(B * N, DF),
         coord.reshape(B * N, 3),
         jnp.zeros((B * N, DX - DF - 3), jnp.float32)], axis=1)
    idx2d = nidx.reshape(P // SC_CH, SC_CH)
    gath = _run_sc_gather(tbl, idx2d)

    crep = jnp.pad(
        jnp.repeat(center_coord.reshape(B * G, 3), K, axis=0),
        ((0, 0), (0, K - 3)))

    M, s = _run_stats(gath, crep)
    s = s[0]
    W0T = jnp.concatenate(
        [W0[:, 3:131].T, W0[:, 0:3].T,
         jnp.zeros((DC - 131, D1), jnp.float32)], axis=0)
    sw = s @ W0T
    mu1 = sw / P + b0
    Ey2 = jnp.sum((M @ W0T) * W0T, 0) / P + 2.0 * b0 * sw / P + b0 * b0
    var1 = Ey2 - mu1 * mu1
    a0 = g0 / jnp.sqrt(var1 + 1e-5)
    c0 = be0 - a0 * mu1
    w0s = W0T * a0[None, :]
    b0s = (a0 * b0 + c0)[None, :]

    km, ssum, ssq = _run_mlp(gath, crep, w0s, b0s, W1.T, b1[None, :])
    m2 = ssum[0] / P
    v2 = ssq[0] / P - m2 * m2
    a2 = g1 / jnp.sqrt(v2 + 1e-5)
    c2 = be1 - a2 * m2

    out = _run_final(km, a2[None, :], c2[None, :])
    return out.reshape(B, G, D2), center_coord


# speedup vs baseline: 13.7217x; 1.2620x over previous
import functools

import jax
import jax.numpy as jnp
from jax import lax
from jax.experimental import pallas as pl
from jax.experimental.pallas import tpu as pltpu
from jax.experimental.pallas import tpu_sc as plsc

B = 8
N = 4096
G = 1024
K = 16
DF = 128
DX = 256
DC = 144
D1 = 256
D2 = 256
P = B * G * K

NC, NS = 2, 16
NW = NC * NS
BPW = P // NW
SC_CH = 128
SC_NCH = BPW // SC_CH


def _fps_body(pr, cc_r, d_ref):
    iota_n = lax.broadcasted_iota(jnp.int32, (B, N), 1)
    iota_g = lax.broadcasted_iota(jnp.int32, (3 * B, G), 1)
    d_ref[...] = jnp.full((B, N), 1e10, jnp.float32)
    cc_r[...] = jnp.zeros((3 * B, G), jnp.float32)
    XYZ = pr[...]

    def step(i, far):
        m2 = lax.broadcasted_iota(jnp.int32, (3 * B, N), 1) == \
            jnp.concatenate([far, far, far], 0)
        csum = jnp.sum(jnp.where(m2, XYZ, 0.0), 1, keepdims=True)
        cc_r[...] = jnp.where(iota_g == i, csum, cc_r[...])
        d = XYZ - csum
        sq = d * d
        dist = (sq[0:B] + sq[2 * B:3 * B]) + sq[B:2 * B]
        D = jnp.minimum(d_ref[...], dist)
        d_ref[...] = D
        mx = jnp.max(D, 1, keepdims=True)
        farn = jnp.min(jnp.where(D == mx, iota_n, N), 1, keepdims=True)
        return farn

    lax.fori_loop(0, G, step, jnp.zeros((B, 1), jnp.int32))


def _run_fps(pstk):
    return pl.pallas_call(
        _fps_body,
        grid=(1,),
        in_specs=[pl.BlockSpec((3 * B, N), lambda i: (0, 0))],
        out_specs=pl.BlockSpec((3 * B, G), lambda i: (0, 0)),
        out_shape=jax.ShapeDtypeStruct((3 * B, G), jnp.float32),
        scratch_shapes=[pltpu.VMEM((B, N), jnp.float32)],
    )(pstk)


GT = 128


def _knn_body(q_r, p_r, out_r, dist_r):
    b = pl.program_id(0)
    q = q_r[0]
    p = p_r[0]
    qp = lax.dot_general(q, p, (((1,), (0,)), ((), ())),
                         preferred_element_type=jnp.float32)
    px = p[0:1, :]; py = p[1:2, :]; pz = p[2:3, :]
    qx = q[:, 0:1]; qy = q[:, 1:2]; qz = q[:, 2:3]
    sq_p = px * px + py * py + pz * pz
    sq_c = qx * qx + qy * qy + qz * qz
    dist_r[...] = sq_c + sq_p - 2.0 * qp
    iota_f = lax.broadcasted_iota(jnp.int32, (GT, N), 1).astype(jnp.float32)
    base = b * N
    cand = dist_r[...]
    for j in range(K):
        m = jnp.min(cand, 1, keepdims=True)
        sf = jnp.min(jnp.where(cand == m, iota_f, float(N)), 1, keepdims=True)
        out_r[0, :, j:j + 1] = sf.astype(jnp.int32) + base
        cand = jnp.where(iota_f == sf, jnp.inf, cand)


def _run_knn(qpad, ppad):
    return pl.pallas_call(
        _knn_body,
        grid=(B, G // GT),
        in_specs=[
            pl.BlockSpec((1, GT, 4), lambda b, g: (b, g, 0)),
            pl.BlockSpec((1, 4, N), lambda b, g: (b, 0, 0)),
        ],
        out_specs=pl.BlockSpec((1, GT, K), lambda b, g: (b, g, 0)),
        out_shape=jax.ShapeDtypeStruct((B, G, K), jnp.int32),
        scratch_shapes=[pltpu.VMEM((GT, N), jnp.float32)],
    )(qpad, ppad)


def _sc_gather_body(tbl_hbm, idx_hbm, out_hbm, idx_v, rows_v, sem):
    wid = lax.axis_index("s") * NC + lax.axis_index("c")
    row0 = wid * SC_NCH
    pltpu.sync_copy(idx_hbm.at[pl.ds(row0, SC_NCH)], idx_v)

    def chunk(i, carry):
        pltpu.async_copy(tbl_hbm.at[idx_v.at[i]], rows_v, sem).wait()
        pltpu.sync_copy(rows_v, out_hbm.at[pl.ds(wid * BPW + i * SC_CH, SC_CH)])
        return carry

    lax.fori_loop(0, SC_NCH, chunk, 0)


def _run_sc_gather(tbl, idx2d):
    mesh = plsc.VectorSubcoreMesh(core_axis_name="c", subcore_axis_name="s")
    f = functools.partial(
        pl.kernel,
        mesh=mesh,
        out_type=jax.ShapeDtypeStruct((P, DX), jnp.float32),
        scratch_types=[
            pltpu.VMEM((SC_NCH, SC_CH), jnp.int32),
            pltpu.VMEM((SC_CH, DX), jnp.float32),
            pltpu.SemaphoreType.DMA,
        ],
    )(_sc_gather_body)
    return f(tbl, idx2d)


PT = 512
NPT = P // PT


def _stats_body(g_r, c_r, m_r, s_r):
    xf = g_r[:, 0:DF]
    xc = g_r[:, DF:DC] - c_r[...]
    X = jnp.concatenate([xf, xc], axis=1)

    @pl.when(pl.program_id(0) == 0)
    def _():
        m_r[...] = jnp.zeros((DC, DC), jnp.float32)
        s_r[...] = jnp.zeros((1, DC), jnp.float32)

    m_r[...] += lax.dot_general(X, X, (((0,), (0,)), ((), ())),
                                preferred_element_type=jnp.float32)
    s_r[...] += jnp.sum(X, 0, keepdims=True)


def _run_stats(gath, crep):
    return pl.pallas_call(
        _stats_body,
        grid=(NPT,),
        in_specs=[
            pl.BlockSpec((PT, DX), lambda i: (i, 0)),
            pl.BlockSpec((PT, K), lambda i: (i, 0)),
        ],
        out_specs=[
            pl.BlockSpec((DC, DC), lambda i: (0, 0)),
            pl.BlockSpec((1, DC), lambda i: (0, 0)),
        ],
        out_shape=[
            jax.ShapeDtypeStruct((DC, DC), jnp.float32),
            jax.ShapeDtypeStruct((1, DC), jnp.float32),
        ],
    )(gath, crep)


def _mlp_body(g_r, c_r, w0_r, b0_r, w1_r, b1_r, km_r, ss_r, sq_r):
    xf = g_r[:, 0:DF]
    xc = g_r[:, DF:DC] - c_r[...]
    X = jnp.concatenate([xf, xc], axis=1)
    y1 = lax.dot_general(X, w0_r[...], (((1,), (0,)), ((), ())),
                         preferred_element_type=jnp.float32) + b0_r[...]
    x2 = jnp.maximum(y1, 0.0)
    y2 = lax.dot_general(x2, w1_r[...], (((1,), (0,)), ((), ())),
                         preferred_element_type=jnp.float32) + b1_r[...]

    @pl.when(pl.program_id(0) == 0)
    def _():
        ss_r[...] = jnp.zeros((1, D2), jnp.float32)
        sq_r[...] = jnp.zeros((1, D2), jnp.float32)

    ss_r[...] += jnp.sum(y2, 0, keepdims=True)
    sq_r[...] += jnp.sum(y2 * y2, 0, keepdims=True)
    km_r[...] = jnp.max(y2.reshape(PT // K, K, D2), axis=1)


def _run_mlp(gath, crep, w0s, b0s, w1t, b1):
    return pl.pallas_call(
        _mlp_body,
        grid=(NPT,),
        in_specs=[
            pl.BlockSpec((PT, DX), lambda i: (i, 0)),
            pl.BlockSpec((PT, K), lambda i: (i, 0)),
            pl.BlockSpec((DC, D1), lambda i: (0, 0)),
            pl.BlockSpec((1, D1), lambda i: (0, 0)),
            pl.BlockSpec((D1, D2), lambda i: (0, 0)),
            pl.BlockSpec((1, D2), lambda i: (0, 0)),
        ],
        out_specs=[
            pl.BlockSpec((PT // K, D2), lambda i: (i, 0)),
            pl.BlockSpec((1, D2), lambda i: (0, 0)),
            pl.BlockSpec((1, D2), lambda i: (0, 0)),
        ],
        out_shape=[
            jax.ShapeDtypeStruct((B * G, D2), jnp.float32),
            jax.ShapeDtypeStruct((1, D2), jnp.float32),
            jax.ShapeDtypeStruct((1, D2), jnp.float32),
        ],
    )(gath, crep, w0s, b0s, w1t, b1)


FT = 2048


def _final_body(km_r, a_r, c_r, o_r):
    o_r[...] = jnp.maximum(a_r[...] * km_r[...] + c_r[...], 0.0)


def _run_final(km, a2, c2):
    return pl.pallas_call(
        _final_body,
        grid=(B * G // FT,),
        in_specs=[
            pl.BlockSpec((FT, D2), lambda i: (i, 0)),
            pl.BlockSpec((1, D2), lambda i: (0, 0)),
            pl.BlockSpec((1, D2), lambda i: (0, 0)),
        ],
        out_specs=pl.BlockSpec((FT, D2), lambda i: (i, 0)),
        out_shape=jax.ShapeDtypeStruct((B * G, D2), jnp.float32),
    )(km, a2, c2)


def kernel(feature, coord, W0, b0, g0, be0, W1, b1, g1, be1):
    pstk = jnp.concatenate(
        [coord[..., 0], coord[..., 1], coord[..., 2]], axis=0)
    ccs = _run_fps(pstk)
    cx, cy, cz = ccs[0:B], ccs[B:2 * B], ccs[2 * B:3 * B]
    center_coord = jnp.stack([cx, cy, cz], axis=-1)

    qpad = jnp.stack([cx, cy, cz, jnp.zeros_like(cx)], axis=-1)
    ppad = jnp.concatenate(
        [coord.transpose(0, 2, 1),
         jnp.zeros((B, 1, N), jnp.float32)], axis=1)
    nidx = _run_knn(qpad, ppad)

    tbl = jnp.concatenate(
        [feature.reshape(B * N, DF),
         coord.reshape(B * N, 3),
         jnp.zeros((B * N, DX - DF - 3), jnp.float32)], axis=1)
    idx2d = nidx.reshape(P // SC_CH, SC_CH)
    gath = _run_sc_gather(tbl, idx2d)

    crep = jnp.pad(
        jnp.repeat(center_coord.reshape(B * G, 3), K, axis=0),
        ((0, 0), (0, K - 3)))

    M, s = _run_stats(gath, crep)
    s = s[0]
    W0T = jnp.concatenate(
        [W0[:, 3:131].T, W0[:, 0:3].T,
         jnp.zeros((DC - 131, D1), jnp.float32)], axis=0)
    sw = s @ W0T
    mu1 = sw / P + b0
    Ey2 = jnp.sum((M @ W0T) * W0T, 0) / P + 2.0 * b0 * sw / P + b0 * b0
    var1 = Ey2 - mu1 * mu1
    a0 = g0 / jnp.sqrt(var1 + 1e-5)
    c0 = be0 - a0 * mu1
    w0s = W0T * a0[None, :]
    b0s = (a0 * b0 + c0)[None, :]

    km, ssum, ssq = _run_mlp(gath, crep, w0s, b0s, W1.T, b1[None, :])
    m2 = ssum[0] / P
    v2 = ssq[0] / P - m2 * m2
    a2 = g1 / jnp.sqrt(v2 + 1e-5)
    c2 = be1 - a2 * m2

    out = _run_final(km, a2[None, :], c2[None, :])
    return out.reshape(B, G, D2), center_coord


# speedup vs baseline: 14.8741x; 1.0840x over previous
import functools

import jax
import jax.numpy as jnp
from jax import lax
from jax.experimental import pallas as pl
from jax.experimental.pallas import tpu as pltpu
from jax.experimental.pallas import tpu_sc as plsc

B = 8
N = 4096
G = 1024
K = 16
DF = 128
DX = 256
DC = 144
D1 = 256
D2 = 256
P = B * G * K

NC, NS = 2, 16
NW = NC * NS
BPW = P // NW
SC_CH = 128
SC_NCH = BPW // SC_CH


def _fps_body(pr, cc_r):
    iota_nf = lax.broadcasted_iota(jnp.int32, (B, N), 1).astype(jnp.float32)
    iota_g = lax.broadcasted_iota(jnp.int32, (3 * B, G), 1)
    cc_r[...] = jnp.zeros((3 * B, G), jnp.float32)
    XYZ = pr[...]
    X = XYZ[0:B]
    Y = XYZ[B:2 * B]
    Z = XYZ[2 * B:3 * B]

    def step(i, carry):
        far, Dp = carry
        m2 = iota_nf == far
        cx = jnp.sum(jnp.where(m2, X, 0.0), 1, keepdims=True)
        cy = jnp.sum(jnp.where(m2, Y, 0.0), 1, keepdims=True)
        cz = jnp.sum(jnp.where(m2, Z, 0.0), 1, keepdims=True)
        csum = jnp.concatenate([cx, cy, cz], 0)
        cc_r[...] = jnp.where(iota_g == i, csum, cc_r[...])
        dx = X - cx
        dy = Y - cy
        dz = Z - cz
        dist = (dx * dx + dz * dz) + dy * dy
        D = jnp.minimum(Dp, dist)
        mx = jnp.max(D, 1, keepdims=True)
        farn = jnp.min(jnp.where(D == mx, iota_nf, float(N)), 1, keepdims=True)
        return farn, D

    lax.fori_loop(0, G, step, (jnp.zeros((B, 1), jnp.float32),
                               jnp.full((B, N), 1e10, jnp.float32)))


def _run_fps(pstk):
    return pl.pallas_call(
        _fps_body,
        grid=(1,),
        in_specs=[pl.BlockSpec((3 * B, N), lambda i: (0, 0))],
        out_specs=pl.BlockSpec((3 * B, G), lambda i: (0, 0)),
        out_shape=jax.ShapeDtypeStruct((3 * B, G), jnp.float32),
    )(pstk)


GT = 128


def _knn_body(q_r, p_r, out_r, dist_r):
    b = pl.program_id(0)
    q = q_r[0]
    p = p_r[0]
    qp = lax.dot_general(q, p, (((1,), (0,)), ((), ())),
                         preferred_element_type=jnp.float32)
    px = p[0:1, :]; py = p[1:2, :]; pz = p[2:3, :]
    qx = q[:, 0:1]; qy = q[:, 1:2]; qz = q[:, 2:3]
    sq_p = px * px + py * py + pz * pz
    sq_c = qx * qx + qy * qy + qz * qz
    dist_r[...] = sq_c + sq_p - 2.0 * qp
    iota_f = lax.broadcasted_iota(jnp.int32, (GT, N), 1).astype(jnp.float32)
    base = b * N
    cand = dist_r[...]
    for j in range(K):
        m = jnp.min(cand, 1, keepdims=True)
        sf = jnp.min(jnp.where(cand == m, iota_f, float(N)), 1, keepdims=True)
        out_r[0, :, j:j + 1] = sf.astype(jnp.int32) + base
        cand = jnp.where(iota_f == sf, jnp.inf, cand)


def _run_knn(qpad, ppad):
    return pl.pallas_call(
        _knn_body,
        grid=(B, G // GT),
        in_specs=[
            pl.BlockSpec((1, GT, 4), lambda b, g: (b, g, 0)),
            pl.BlockSpec((1, 4, N), lambda b, g: (b, 0, 0)),
        ],
        out_specs=pl.BlockSpec((1, GT, K), lambda b, g: (b, g, 0)),
        out_shape=jax.ShapeDtypeStruct((B, G, K), jnp.int32),
        scratch_shapes=[pltpu.VMEM((GT, N), jnp.float32)],
    )(qpad, ppad)


def _sc_gather_body(tbl_hbm, idx_hbm, out_hbm, idx_v, rows_v, sem):
    wid = lax.axis_index("s") * NC + lax.axis_index("c")
    row0 = wid * SC_NCH
    pltpu.sync_copy(idx_hbm.at[pl.ds(row0, SC_NCH)], idx_v)

    def chunk(i, carry):
        pltpu.async_copy(tbl_hbm.at[idx_v.at[i]], rows_v, sem).wait()
        pltpu.sync_copy(rows_v, out_hbm.at[pl.ds(wid * BPW + i * SC_CH, SC_CH)])
        return carry

    lax.fori_loop(0, SC_NCH, chunk, 0)


def _run_sc_gather(tbl, idx2d):
    mesh = plsc.VectorSubcoreMesh(core_axis_name="c", subcore_axis_name="s")
    f = functools.partial(
        pl.kernel,
        mesh=mesh,
        out_type=jax.ShapeDtypeStruct((P, DX), jnp.float32),
        scratch_types=[
            pltpu.VMEM((SC_NCH, SC_CH), jnp.int32),
            pltpu.VMEM((SC_CH, DX), jnp.float32),
            pltpu.SemaphoreType.DMA,
        ],
    )(_sc_gather_body)
    return f(tbl, idx2d)


PT = 512
NPT = P // PT


def _stats_body(g_r, c_r, m_r, s_r):
    xf = g_r[:, 0:DF]
    xc = g_r[:, DF:DC] - c_r[...]
    X = jnp.concatenate([xf, xc], axis=1)

    @pl.when(pl.program_id(0) == 0)
    def _():
        m_r[...] = jnp.zeros((DC, DC), jnp.float32)
        s_r[...] = jnp.zeros((1, DC), jnp.float32)

    m_r[...] += lax.dot_general(X, X, (((0,), (0,)), ((), ())),
                                preferred_element_type=jnp.float32)
    s_r[...] += jnp.sum(X, 0, keepdims=True)


def _run_stats(gath, crep):
    return pl.pallas_call(
        _stats_body,
        grid=(NPT,),
        in_specs=[
            pl.BlockSpec((PT, DX), lambda i: (i, 0)),
            pl.BlockSpec((PT, K), lambda i: (i, 0)),
        ],
        out_specs=[
            pl.BlockSpec((DC, DC), lambda i: (0, 0)),
            pl.BlockSpec((1, DC), lambda i: (0, 0)),
        ],
        out_shape=[
            jax.ShapeDtypeStruct((DC, DC), jnp.float32),
            jax.ShapeDtypeStruct((1, DC), jnp.float32),
        ],
    )(gath, crep)


def _mlp_body(g_r, c_r, w0_r, b0_r, w1_r, b1_r, km_r, ss_r, sq_r):
    xf = g_r[:, 0:DF]
    xc = g_r[:, DF:DC] - c_r[...]
    X = jnp.concatenate([xf, xc], axis=1)
    y1 = lax.dot_general(X, w0_r[...], (((1,), (0,)), ((), ())),
                         preferred_element_type=jnp.float32) + b0_r[...]
    x2 = jnp.maximum(y1, 0.0)
    y2 = lax.dot_general(x2, w1_r[...], (((1,), (0,)), ((), ())),
                         preferred_element_type=jnp.float32) + b1_r[...]

    @pl.when(pl.program_id(0) == 0)
    def _():
        ss_r[...] = jnp.zeros((1, D2), jnp.float32)
        sq_r[...] = jnp.zeros((1, D2), jnp.float32)

    ss_r[...] += jnp.sum(y2, 0, keepdims=True)
    sq_r[...] += jnp.sum(y2 * y2, 0, keepdims=True)
    km_r[...] = jnp.max(y2.reshape(PT // K, K, D2), axis=1)


def _run_mlp(gath, crep, w0s, b0s, w1t, b1):
    return pl.pallas_call(
        _mlp_body,
        grid=(NPT,),
        in_specs=[
            pl.BlockSpec((PT, DX), lambda i: (i, 0)),
            pl.BlockSpec((PT, K), lambda i: (i, 0)),
            pl.BlockSpec((DC, D1), lambda i: (0, 0)),
            pl.BlockSpec((1, D1), lambda i: (0, 0)),
            pl.BlockSpec((D1, D2), lambda i: (0, 0)),
            pl.BlockSpec((1, D2), lambda i: (0, 0)),
        ],
        out_specs=[
            pl.BlockSpec((PT // K, D2), lambda i: (i, 0)),
            pl.BlockSpec((1, D2), lambda i: (0, 0)),
            pl.BlockSpec((1, D2), lambda i: (0, 0)),
        ],
        out_shape=[
            jax.ShapeDtypeStruct((B * G, D2), jnp.float32),
            jax.ShapeDtypeStruct((1, D2), jnp.float32),
            jax.ShapeDtypeStruct((1, D2), jnp.float32),
        ],
    )(gath, crep, w0s, b0s, w1t, b1)


FT = 2048


def _final_body(km_r, a_r, c_r, o_r):
    o_r[...] = jnp.maximum(a_r[...] * km_r[...] + c_r[...], 0.0)


def _run_final(km, a2, c2):
    return pl.pallas_call(
        _final_body,
        grid=(B * G // FT,),
        in_specs=[
            pl.BlockSpec((FT, D2), lambda i: (i, 0)),
            pl.BlockSpec((1, D2), lambda i: (0, 0)),
            pl.BlockSpec((1, D2), lambda i: (0, 0)),
        ],
        out_specs=pl.BlockSpec((FT, D2), lambda i: (i, 0)),
        out_shape=jax.ShapeDtypeStruct((B * G, D2), jnp.float32),
    )(km, a2, c2)


def kernel(feature, coord, W0, b0, g0, be0, W1, b1, g1, be1):
    pstk = jnp.concatenate(
        [coord[..., 0], coord[..., 1], coord[..., 2]], axis=0)
    ccs = _run_fps(pstk)
    cx, cy, cz = ccs[0:B], ccs[B:2 * B], ccs[2 * B:3 * B]
    center_coord = jnp.stack([cx, cy, cz], axis=-1)

    qpad = jnp.stack([cx, cy, cz, jnp.zeros_like(cx)], axis=-1)
    ppad = jnp.concatenate(
        [coord.transpose(0, 2, 1),
         jnp.zeros((B, 1, N), jnp.float32)], axis=1)
    nidx = _run_knn(qpad, ppad)

    tbl = jnp.concatenate(
        [feature.reshape(B * N, DF),
         coord.reshape(B * N, 3),
         jnp.zeros((B * N, DX - DF - 3), jnp.float32)], axis=1)
    idx2d = nidx.reshape(P // SC_CH, SC_CH)
    gath = _run_sc_gather(tbl, idx2d)

    crep = jnp.pad(
        jnp.repeat(center_coord.reshape(B * G, 3), K, axis=0),
        ((0, 0), (0, K - 3)))

    M, s = _run_stats(gath, crep)
    s = s[0]
    W0T = jnp.concatenate(
        [W0[:, 3:131].T, W0[:, 0:3].T,
         jnp.zeros((DC - 131, D1), jnp.float32)], axis=0)
    sw = s @ W0T
    mu1 = sw / P + b0
    Ey2 = jnp.sum((M @ W0T) * W0T, 0) / P + 2.0 * b0 * sw / P + b0 * b0
    var1 = Ey2 - mu1 * mu1
    a0 = g0 / jnp.sqrt(var1 + 1e-5)
    c0 = be0 - a0 * mu1
    w0s = W0T * a0[None, :]
    b0s = (a0 * b0 + c0)[None, :]

    km, ssum, ssq = _run_mlp(gath, crep, w0s, b0s, W1.T, b1[None, :])
    m2 = ssum[0] / P
    v2 = ssq[0] / P - m2 * m2
    a2 = g1 / jnp.sqrt(v2 + 1e-5)
    c2 = be1 - a2 * m2

    out = _run_final(km, a2[None, :], c2[None, :])
    return out.reshape(B, G, D2), center_coord
